# baseline (speedup 1.0000x reference)
import functools

import jax
import jax.numpy as jnp
from jax import lax
from jax.experimental import pallas as pl
from jax.experimental.pallas import tpu as pltpu
from jax.experimental.pallas import tpu_sc as plsc

_G = 32
_D = 768
_HALF = _D // 2
_BATCH = 64
_NW = 32
_FIRE = 16


def _sc_body(row_hbm, col_hbm, out_hbm, tile_v, sem):
    wid = lax.axis_index("s") * 2 + lax.axis_index("c")
    pltpu.sync_copy(col_hbm, tile_v.at[0, :, pl.ds(_HALF, _HALF)])
    for j in range(_G):
        pltpu.sync_copy(
            row_hbm.at[pl.ds(wid, 1)],
            tile_v.at[0, pl.ds(j, 1), pl.ds(0, _HALF)],
        )
    for b0 in range(0, _BATCH, _FIRE):
        copies = [
            pltpu.async_copy(
                tile_v, out_hbm.at[pl.ds((b0 + k) * _G + wid, 1)], sem
            )
            for k in range(_FIRE)
        ]
        for c in copies:
            c.wait()


def kernel(batch_size, row_table, col_table):
    del batch_size
    mesh = plsc.VectorSubcoreMesh(core_axis_name="c", subcore_axis_name="s")
    run = functools.partial(
        pl.kernel,
        mesh=mesh,
        out_type=jax.ShapeDtypeStruct((_BATCH * _G, _G, _D), jnp.float32),
        scratch_types=[
            pltpu.VMEM((1, _G, _D), jnp.float32),
            pltpu.SemaphoreType.DMA,
        ],
    )(_sc_body)
    out = run(row_table, col_table)
    return out.reshape(_BATCH, _G, _G, _D)


# speedup vs baseline: 1.0013x; 1.0013x over previous
import functools

import jax
import jax.numpy as jnp
from jax import lax
from jax.experimental import pallas as pl
from jax.experimental.pallas import tpu as pltpu
from jax.experimental.pallas import tpu_sc as plsc

_G = 32
_D = 768
_HALF = _D // 2
_BATCH = 64
_BB = 2


def _sc_gather_body(row_hbm, col_hbm, pos_hbm, tile_v, sem):
    wid = lax.axis_index("s") * 2 + lax.axis_index("c")
    pltpu.sync_copy(col_hbm, tile_v.at[0, :, pl.ds(_HALF, _HALF)])
    for j in range(_G):
        pltpu.sync_copy(
            row_hbm.at[pl.ds(wid, 1)],
            tile_v.at[0, pl.ds(j, 1), pl.ds(0, _HALF)],
        )
    pltpu.async_copy(tile_v, pos_hbm.at[pl.ds(wid, 1)], sem).wait()


def _tc_broadcast_body(pos_ref, out_ref):
    out_ref[...] = jnp.broadcast_to(pos_ref[...][None], (_BB, _G, _G, _D))


def kernel(batch_size, row_table, col_table):
    del batch_size
    mesh = plsc.VectorSubcoreMesh(core_axis_name="c", subcore_axis_name="s")
    sc_gather = functools.partial(
        pl.kernel,
        mesh=mesh,
        out_type=jax.ShapeDtypeStruct((_G, _G, _D), jnp.float32),
        scratch_types=[
            pltpu.VMEM((1, _G, _D), jnp.float32),
            pltpu.SemaphoreType.DMA,
        ],
    )(_sc_gather_body)
    pos_emb = sc_gather(row_table, col_table)

    return pl.pallas_call(
        _tc_broadcast_body,
        grid=(_BATCH // _BB,),
        in_specs=[pl.BlockSpec((_G, _G, _D), lambda b: (0, 0, 0))],
        out_specs=pl.BlockSpec((_BB, _G, _G, _D), lambda b: (b, 0, 0, 0)),
        out_shape=jax.ShapeDtypeStruct((_BATCH, _G, _G, _D), jnp.float32),
    )(pos_emb)


# speedup vs baseline: 1.1836x; 1.1820x over previous
import functools

import jax
import jax.numpy as jnp
from jax import lax
from jax.experimental import pallas as pl
from jax.experimental.pallas import tpu as pltpu
from jax.experimental.pallas import tpu_sc as plsc

_G = 32
_D = 768
_HALF = _D // 2
_LANES = 16
_BATCH = 64
_BB = 2


def _sc_gather_body(row_hbm, col_hbm, pos_hbm, tile_v, row_v, sem):
    wid = lax.axis_index("s") * 2 + lax.axis_index("c")
    col_copy = pltpu.async_copy(col_hbm, tile_v.at[0, :, pl.ds(_HALF, _HALF)], sem)
    pltpu.sync_copy(row_hbm.at[pl.ds(wid, 1)], row_v)
    for k in range(_HALF // _LANES):
        v = row_v[0, pl.ds(k * _LANES, _LANES)]
        for j in range(_G):
            tile_v[0, j, pl.ds(k * _LANES, _LANES)] = v
    col_copy.wait()
    pltpu.async_copy(tile_v, pos_hbm.at[pl.ds(wid, 1)], sem).wait()


def _tc_broadcast_body(pos_ref, out_ref):
    out_ref[...] = jnp.broadcast_to(pos_ref[...][None], (_BB, _G, _G, _D))


def kernel(batch_size, row_table, col_table):
    del batch_size
    mesh = plsc.VectorSubcoreMesh(core_axis_name="c", subcore_axis_name="s")
    sc_gather = functools.partial(
        pl.kernel,
        mesh=mesh,
        out_type=jax.ShapeDtypeStruct((_G, _G, _D), jnp.float32),
        scratch_types=[
            pltpu.VMEM((1, _G, _D), jnp.float32),
            pltpu.VMEM((1, _HALF), jnp.float32),
            pltpu.SemaphoreType.DMA,
        ],
    )(_sc_gather_body)
    pos_emb = sc_gather(row_table, col_table)

    return pl.pallas_call(
        _tc_broadcast_body,
        grid=(_BATCH // _BB,),
        in_specs=[pl.BlockSpec((_G, _G, _D), lambda b: (0, 0, 0))],
        out_specs=pl.BlockSpec((_BB, _G, _G, _D), lambda b: (b, 0, 0, 0)),
        out_shape=jax.ShapeDtypeStruct((_BATCH, _G, _G, _D), jnp.float32),
    )(pos_emb)


# speedup vs baseline: 1.3180x; 1.1136x over previous
import functools

import jax
import jax.numpy as jnp
from jax import lax
from jax.experimental import pallas as pl
from jax.experimental.pallas import tpu as pltpu
from jax.experimental.pallas import tpu_sc as plsc

_G = 32
_D = 768
_HALF = _D // 2
_BATCH = 64
_BB = 2
_TAIL = _BB
_HEAD = _BATCH - _TAIL


def _sc_concat_body(row_hbm, col_hbm, rowcol_hbm):
    wid = lax.axis_index("s") * 2 + lax.axis_index("c")

    @pl.when(wid == 0)
    def _():
        pltpu.sync_copy(row_hbm, rowcol_hbm.at[:, pl.ds(0, _HALF)])

    @pl.when(wid == 1)
    def _():
        pltpu.sync_copy(col_hbm, rowcol_hbm.at[:, pl.ds(_HALF, _HALF)])


def _tc_head_body(row_ref, col_ref, out_ref):
    r = row_ref[...]
    c = col_ref[...]
    re = jnp.broadcast_to(r[:, None, :], (_G, _G, _HALF))
    ce = jnp.broadcast_to(c[None, :, :], (_G, _G, _HALF))
    pos = jnp.concatenate([re, ce], axis=-1)
    out_ref[...] = jnp.broadcast_to(pos[None], (_BB, _G, _G, _D))


def _tc_tail_body(rowcol_ref, part_ref, out_ref):
    del part_ref
    rc = rowcol_ref[...]
    r = rc[:, :_HALF]
    c = rc[:, _HALF:]
    re = jnp.broadcast_to(r[:, None, :], (_G, _G, _HALF))
    ce = jnp.broadcast_to(c[None, :, :], (_G, _G, _HALF))
    pos = jnp.concatenate([re, ce], axis=-1)
    out_ref[...] = jnp.broadcast_to(pos[None], (_TAIL, _G, _G, _D))


def kernel(batch_size, row_table, col_table):
    del batch_size
    mesh = plsc.VectorSubcoreMesh(core_axis_name="c", subcore_axis_name="s")
    sc_concat = functools.partial(
        pl.kernel,
        mesh=mesh,
        out_type=jax.ShapeDtypeStruct((_G, _D), jnp.float32),
    )(_sc_concat_body)
    part = pl.pallas_call(
        _tc_head_body,
        grid=(_HEAD // _BB,),
        in_specs=[
            pl.BlockSpec((_G, _HALF), lambda b: (0, 0)),
            pl.BlockSpec((_G, _HALF), lambda b: (0, 0)),
        ],
        out_specs=pl.BlockSpec((_BB, _G, _G, _D), lambda b: (b, 0, 0, 0)),
        out_shape=jax.ShapeDtypeStruct((_BATCH, _G, _G, _D), jnp.float32),
    )(row_table, col_table)

    rowcol = sc_concat(row_table, col_table)

    return pl.pallas_call(
        _tc_tail_body,
        grid=(1,),
        in_specs=[
            pl.BlockSpec((_G, _D), lambda b: (0, 0)),
            pl.BlockSpec(memory_space=pltpu.MemorySpace.HBM),
        ],
        out_specs=pl.BlockSpec((_TAIL, _G, _G, _D), lambda b: (_HEAD // _TAIL, 0, 0, 0)),
        out_shape=jax.ShapeDtypeStruct((_BATCH, _G, _G, _D), jnp.float32),
        input_output_aliases={1: 0},
    )(rowcol, part)


# speedup vs baseline: 1.3200x; 1.0015x over previous
import functools

import jax
import jax.numpy as jnp
from jax import lax
from jax.experimental import pallas as pl
from jax.experimental.pallas import tpu as pltpu
from jax.experimental.pallas import tpu_sc as plsc

_G = 32
_D = 768
_HALF = _D // 2
_BATCH = 64
_BB = 2
_TAIL = _BB
_HEAD = _BATCH - _TAIL


def _sc_concat_body(row_hbm, col_hbm, rowcol_hbm):
    cid = lax.axis_index("c")

    @pl.when(cid == 0)
    def _():
        pltpu.sync_copy(row_hbm, rowcol_hbm.at[:, pl.ds(0, _HALF)])

    @pl.when(cid == 1)
    def _():
        pltpu.sync_copy(col_hbm, rowcol_hbm.at[:, pl.ds(_HALF, _HALF)])


def _tc_head_body(row_ref, col_ref, out_ref):
    r = row_ref[...]
    c = col_ref[...]
    re = jnp.broadcast_to(r[:, None, :], (_G, _G, _HALF))
    ce = jnp.broadcast_to(c[None, :, :], (_G, _G, _HALF))
    pos = jnp.concatenate([re, ce], axis=-1)
    out_ref[...] = jnp.broadcast_to(pos[None], (_BB, _G, _G, _D))


def _tc_tail_body(rowcol_ref, part_ref, out_ref):
    del part_ref
    rc = rowcol_ref[...]
    r = rc[:, :_HALF]
    c = rc[:, _HALF:]
    re = jnp.broadcast_to(r[:, None, :], (_G, _G, _HALF))
    ce = jnp.broadcast_to(c[None, :, :], (_G, _G, _HALF))
    pos = jnp.concatenate([re, ce], axis=-1)
    out_ref[...] = jnp.broadcast_to(pos[None], (_TAIL, _G, _G, _D))


def kernel(batch_size, row_table, col_table):
    del batch_size
    mesh = plsc.ScalarSubcoreMesh(axis_name="c", num_cores=2)
    sc_concat = functools.partial(
        pl.kernel,
        mesh=mesh,
        out_type=jax.ShapeDtypeStruct((_G, _D), jnp.float32),
    )(_sc_concat_body)
    part = pl.pallas_call(
        _tc_head_body,
        grid=(_HEAD // _BB,),
        in_specs=[
            pl.BlockSpec((_G, _HALF), lambda b: (0, 0)),
            pl.BlockSpec((_G, _HALF), lambda b: (0, 0)),
        ],
        out_specs=pl.BlockSpec((_BB, _G, _G, _D), lambda b: (b, 0, 0, 0)),
        out_shape=jax.ShapeDtypeStruct((_BATCH, _G, _G, _D), jnp.float32),
    )(row_table, col_table)

    rowcol = sc_concat(row_table, col_table)

    return pl.pallas_call(
        _tc_tail_body,
        grid=(1,),
        in_specs=[
            pl.BlockSpec((_G, _D), lambda b: (0, 0)),
            pl.BlockSpec(memory_space=pltpu.MemorySpace.HBM),
        ],
        out_specs=pl.BlockSpec((_TAIL, _G, _G, _D), lambda b: (_HEAD // _TAIL, 0, 0, 0)),
        out_shape=jax.ShapeDtypeStruct((_BATCH, _G, _G, _D), jnp.float32),
        input_output_aliases={1: 0},
    )(rowcol, part)


# speedup vs baseline: 1.3362x; 1.0123x over previous
import functools

import jax
import jax.numpy as jnp
from jax import lax
from jax.experimental import pallas as pl
from jax.experimental.pallas import tpu as pltpu
from jax.experimental.pallas import tpu_sc as plsc

_G = 32
_D = 768
_HALF = _D // 2
_BATCH = 64
_BB = 2
_TAIL = _BB
_HEAD = _BATCH - _TAIL


def _sc_concat_body(row_hbm, col_hbm, rowcol_hbm):
    cid = lax.axis_index("c")

    @pl.when(cid == 0)
    def _():
        pltpu.sync_copy(row_hbm, rowcol_hbm.at[:, pl.ds(0, _HALF)])

    @pl.when(cid == 1)
    def _():
        pltpu.sync_copy(col_hbm, rowcol_hbm.at[:, pl.ds(_HALF, _HALF)])


def _tc_head_body(row_ref, col_ref, out_ref):
    r = row_ref[...]
    c = col_ref[...]
    re = jnp.broadcast_to(r[:, None, :], (_G, _G, _HALF))
    ce = jnp.broadcast_to(c[None, :, :], (_G, _G, _HALF))
    pos = jnp.concatenate([re, ce], axis=-1)
    out_ref[...] = jnp.broadcast_to(pos[None], (_BB, _G, _G, _D))


def _tc_tail_body(rowcol_ref, part_ref, out_ref):
    del part_ref
    rc = rowcol_ref[...]
    r = rc[:, :_HALF]
    c = rc[:, _HALF:]
    re = jnp.broadcast_to(r[:, None, :], (_G, _G, _HALF))
    ce = jnp.broadcast_to(c[None, :, :], (_G, _G, _HALF))
    pos = jnp.concatenate([re, ce], axis=-1)
    out_ref[...] = pos[None]


def kernel(batch_size, row_table, col_table):
    del batch_size
    mesh = plsc.ScalarSubcoreMesh(axis_name="c", num_cores=2)
    sc_concat = functools.partial(
        pl.kernel,
        mesh=mesh,
        out_type=jax.ShapeDtypeStruct((_G, _D), jnp.float32),
    )(_sc_concat_body)
    part = pl.pallas_call(
        _tc_head_body,
        grid=(_HEAD // _BB,),
        in_specs=[
            pl.BlockSpec((_G, _HALF), lambda b: (0, 0)),
            pl.BlockSpec((_G, _HALF), lambda b: (0, 0)),
        ],
        out_specs=pl.BlockSpec((_BB, _G, _G, _D), lambda b: (b, 0, 0, 0)),
        out_shape=jax.ShapeDtypeStruct((_BATCH, _G, _G, _D), jnp.float32),
    )(row_table, col_table)

    rowcol = sc_concat(row_table, col_table)

    return pl.pallas_call(
        _tc_tail_body,
        grid=(_TAIL,),
        in_specs=[
            pl.BlockSpec((_G, _D), lambda b: (0, 0)),
            pl.BlockSpec(memory_space=pltpu.MemorySpace.HBM),
        ],
        out_specs=pl.BlockSpec((1, _G, _G, _D), lambda b: (_HEAD + b, 0, 0, 0)),
        out_shape=jax.ShapeDtypeStruct((_BATCH, _G, _G, _D), jnp.float32),
        input_output_aliases={1: 0},
    )(rowcol, part)
